# baseline (speedup 1.0000x reference)
import functools

import jax
import jax.numpy as jnp
from jax import lax
from jax.experimental import pallas as pl
from jax.experimental.pallas import tpu as pltpu
from jax.experimental.pallas import tpu_sc as plsc

N = 10000
E = 320000
F_IN = 128
H = 64
G = 64

CHUNK = 128
NCHUNK = E // CHUNK
N_PAD = 10112
NCOPY = 2
ACC_ROWS = NCOPY * N_PAD
ROWS_PER_TILE = ACC_ROWS // 16

CB = 64
NCHUNK_PAD = 2560
IDX_GRID = NCHUNK_PAD // CB

BLK = 2000


def _didx_body(dst_ref, out_ref):
    d = dst_ref[...]
    a = d[:, :, None]
    b = d[:, None, :]
    ii = lax.broadcasted_iota(jnp.int32, (CB, CHUNK, CHUNK), 1)
    jj = lax.broadcasted_iota(jnp.int32, (CB, CHUNK, CHUNK), 2)
    dup = jnp.logical_and(a == b, jj < ii)
    has = jnp.max(dup.astype(jnp.int32), axis=2)
    out_ref[...] = d + has * N_PAD


def _tc_didx(dst2):
    return pl.pallas_call(
        _didx_body,
        grid=(IDX_GRID,),
        in_specs=[pl.BlockSpec((CB, CHUNK), lambda i: (i, 0))],
        out_specs=pl.BlockSpec((CB, CHUNK), lambda i: (i, 0)),
        out_shape=jax.ShapeDtypeStruct((NCHUNK_PAD, CHUNK), jnp.int32),
    )(dst2)


@functools.cache
def _make_sc_segment_sum(fh):
    mesh = plsc.VectorSubcoreMesh(core_axis_name="c", subcore_axis_name="s")

    @functools.partial(
        pl.kernel,
        mesh=mesh,
        compiler_params=pltpu.CompilerParams(use_tc_tiling_on_sc=False),
        out_type=jax.ShapeDtypeStruct((2, ACC_ROWS, fh), jnp.float32),
        scratch_types=[
            pltpu.VMEM((CHUNK,), jnp.int32),
            pltpu.VMEM((CHUNK,), jnp.int32),
            pltpu.VMEM((CHUNK, fh), jnp.float32),
            pltpu.VMEM_SHARED((ACC_ROWS, fh), jnp.float32),
            pltpu.SemaphoreType.DMA,
        ],
    )
    def _sc_segment_sum(table_hbm, src_hbm, didx_hbm, zeros_hbm, out_hbm,
                        si, di, rows, acc, sem):
        c = lax.axis_index("c")
        s = lax.axis_index("s")
        row0 = pl.multiple_of(s * ROWS_PER_TILE, 8)

        pltpu.sync_copy(zeros_hbm.at[pl.ds(row0, ROWS_PER_TILE)],
                        acc.at[pl.ds(row0, ROWS_PER_TILE)])
        plsc.subcore_barrier()

        n_my = jnp.where(s < NCHUNK - 156 * 16, 157, 156)

        def body(k, carry):
            ch = s + k * 16
            e0 = pl.multiple_of(ch * CHUNK, 8)
            pltpu.sync_copy(src_hbm.at[pl.ds(e0, CHUNK)], si)
            pltpu.sync_copy(didx_hbm.at[ch], di)
            pltpu.async_copy(table_hbm.at[c].at[si], rows, sem).wait()
            pltpu.sync_copy(rows, acc.at[di], add=True)
            return carry

        lax.fori_loop(0, n_my, body, 0)

        plsc.subcore_barrier()
        pltpu.sync_copy(acc.at[pl.ds(row0, ROWS_PER_TILE)],
                        out_hbm.at[c, pl.ds(row0, ROWS_PER_TILE)])

    return _sc_segment_sum


def _mlp1_body(x_ref, agg_ref, vec_ref, w1a_ref, w1b_ref, h_ref):
    v = vec_ref[...]
    a = agg_ref[...]
    agg = jnp.concatenate([a[0, 0] + a[0, 1], a[1, 0] + a[1, 1]], axis=1)
    t = x_ref[...] + agg
    t = jnp.dot(t, w1a_ref[...], preferred_element_type=jnp.float32) + v[0]
    t = t / jnp.sqrt(1.0 + 1e-5) * v[1] + v[2]
    t = jnp.maximum(t, 0.0)
    t = jnp.dot(t, w1b_ref[...], preferred_element_type=jnp.float32) + v[3]
    t = jnp.maximum(t, 0.0)
    h_ref[...] = t / jnp.sqrt(1.0 + 1e-5) * v[4] + v[5]


def _tc_mlp1(x, aggp, vecs, w1a, w1b):
    return pl.pallas_call(
        _mlp1_body,
        grid=(N // BLK,),
        in_specs=[
            pl.BlockSpec((BLK, F_IN), lambda i: (i, 0)),
            pl.BlockSpec((2, NCOPY, BLK, H), lambda i: (0, 0, i, 0)),
            pl.BlockSpec((8, H), lambda i: (0, 0)),
            pl.BlockSpec((F_IN, H), lambda i: (0, 0)),
            pl.BlockSpec((H, H), lambda i: (0, 0)),
        ],
        out_specs=pl.BlockSpec((BLK, H), lambda i: (i, 0)),
        out_shape=jax.ShapeDtypeStruct((N, H), jnp.float32),
    )(x, aggp, vecs, w1a, w1b)


def _final_body(h_ref, agg_ref, vec_ref, w2a_ref, w2b_ref, batch_ref,
                wl1_ref, wl2p_ref, out_ref, pool_acc):
    i = pl.program_id(0)
    v = vec_ref[...]
    a = agg_ref[...]
    agg = jnp.concatenate([a[0, 0] + a[0, 1], a[1, 0] + a[1, 1]], axis=1)
    t = h_ref[...] + agg
    t = jnp.dot(t, w2a_ref[...], preferred_element_type=jnp.float32) + v[0]
    t = t / jnp.sqrt(1.0 + 1e-5) * v[1] + v[2]
    t = jnp.maximum(t, 0.0)
    t = jnp.dot(t, w2b_ref[...], preferred_element_type=jnp.float32) + v[3]
    t = jnp.maximum(t, 0.0)
    h2 = t / jnp.sqrt(1.0 + 1e-5) * v[4] + v[5]

    gid = lax.broadcasted_iota(jnp.int32, (BLK, G), 1)
    onehot = jnp.where(batch_ref[...] == gid, 1.0, 0.0).astype(jnp.float32)
    part = lax.dot_general(onehot, h2, (((0,), (0,)), ((), ())),
                           preferred_element_type=jnp.float32,
                           precision=lax.Precision.HIGHEST)

    @pl.when(i == 0)
    def _():
        pool_acc[...] = jnp.zeros_like(pool_acc)

    pool_acc[...] += part

    @pl.when(i == pl.num_programs(0) - 1)
    def _():
        pooled = pool_acc[...]
        p1 = jnp.dot(pooled, wl1_ref[...],
                     preferred_element_type=jnp.float32) + v[6]
        p1 = jnp.maximum(p1, 0.0)
        out = jnp.dot(p1, wl2p_ref[...], preferred_element_type=jnp.float32)
        out_ref[...] = out[:, 0:1] + v[8][0]


def _tc_final(h, aggp, vecs, w2a, w2b, batch2d, wl1, wl2p):
    return pl.pallas_call(
        _final_body,
        grid=(N // BLK,),
        in_specs=[
            pl.BlockSpec((BLK, H), lambda i: (i, 0)),
            pl.BlockSpec((2, NCOPY, BLK, H // 2), lambda i: (0, 0, i, 0)),
            pl.BlockSpec((16, H), lambda i: (0, 0)),
            pl.BlockSpec((H, H), lambda i: (0, 0)),
            pl.BlockSpec((H, H), lambda i: (0, 0)),
            pl.BlockSpec((BLK, 1), lambda i: (i, 0)),
            pl.BlockSpec((H, G), lambda i: (0, 0)),
            pl.BlockSpec((H, 128), lambda i: (0, 0)),
        ],
        out_specs=pl.BlockSpec((G, 1), lambda i: (0, 0)),
        out_shape=jax.ShapeDtypeStruct((G, 1), jnp.float32),
        scratch_shapes=[pltpu.VMEM((G, G), jnp.float32)],
    )(h, aggp, vecs, w2a, w2b, batch2d, wl1, wl2p)


def kernel(x, edge_index, batch,
           w1a, b1a, g1a, be1a, w1b, b1b, g_bn1, be_bn1,
           w2a, b2a, g2a, be2a, w2b, b2b, g_bn2, be_bn2,
           wl1, bl1, wl2, bl2):
    src = edge_index[0]
    dst = edge_index[1]

    dst2 = jnp.concatenate(
        [dst, jnp.zeros((NCHUNK_PAD * CHUNK - E,), jnp.int32)]
    ).reshape(NCHUNK_PAD, CHUNK)
    didx = _tc_didx(dst2)

    vecs1 = jnp.stack([
        b1a, g1a, be1a, b1b, g_bn1, be_bn1,
        jnp.zeros((H,), jnp.float32), jnp.zeros((H,), jnp.float32)])
    vecs2 = jnp.stack([
        b2a, g2a, be2a, b2b, g_bn2, be_bn2,
        bl1, wl2[:, 0], jnp.full((H,), bl2[0], jnp.float32)]
        + [jnp.zeros((H,), jnp.float32)] * 7)

    seg64 = _make_sc_segment_sum(H)
    seg32 = _make_sc_segment_sum(H // 2)

    x_halves = jnp.stack([x[:, :H], x[:, H:]])
    zeros64 = jnp.zeros((ACC_ROWS, H), jnp.float32)
    agg1 = seg64(x_halves, src, didx, zeros64)
    h = _tc_mlp1(x, agg1.reshape(2, NCOPY, N_PAD, H), vecs1, w1a, w1b)

    h_halves = jnp.stack([h[:, :H // 2], h[:, H // 2:]])
    zeros32 = jnp.zeros((ACC_ROWS, H // 2), jnp.float32)
    agg2 = seg32(h_halves, src, didx, zeros32)
    wl2p = jnp.concatenate(
        [wl2, jnp.zeros((H, 127), jnp.float32)], axis=1)
    out = _tc_final(h, agg2.reshape(2, NCOPY, N_PAD, H // 2), vecs2,
                    w2a, w2b, batch.reshape(N, 1), wl1, wl2p)
    return out


# speedup vs baseline: 1.6957x; 1.6957x over previous
import functools

import jax
import jax.numpy as jnp
from jax import lax
from jax.experimental import pallas as pl
from jax.experimental.pallas import tpu as pltpu
from jax.experimental.pallas import tpu_sc as plsc

N = 10000
E = 320000
F_IN = 128
H = 64
G = 64

CHUNK = 128
NCHUNK = E // CHUNK
N_PAD = 10112
NCOPY = 2
ACC_ROWS = NCOPY * N_PAD
ROWS_PER_TILE = ACC_ROWS // 16

CB = 64
NCHUNK_PAD = 2560
IDX_GRID = NCHUNK_PAD // CB

BLK = 2000


def _didx_body(dst_ref, out_ref):
    d = dst_ref[...]
    a = d[:, :, None]
    b = d[:, None, :]
    ii = lax.broadcasted_iota(jnp.int32, (CB, CHUNK, CHUNK), 1)
    jj = lax.broadcasted_iota(jnp.int32, (CB, CHUNK, CHUNK), 2)
    dup = jnp.logical_and(a == b, jj < ii)
    has = jnp.max(dup.astype(jnp.int32), axis=2)
    out_ref[...] = d + has * N_PAD


def _tc_didx(dst2):
    return pl.pallas_call(
        _didx_body,
        grid=(IDX_GRID,),
        in_specs=[pl.BlockSpec((CB, CHUNK), lambda i: (i, 0))],
        out_specs=pl.BlockSpec((CB, CHUNK), lambda i: (i, 0)),
        out_shape=jax.ShapeDtypeStruct((NCHUNK_PAD, CHUNK), jnp.int32),
    )(dst2)


GC = 4
CPT = NCHUNK // 16
NGRP = CPT // GC


@functools.cache
def _make_sc_segment_sum(fh):
    mesh = plsc.VectorSubcoreMesh(core_axis_name="c", subcore_axis_name="s")

    @functools.partial(
        pl.kernel,
        mesh=mesh,
        compiler_params=pltpu.CompilerParams(use_tc_tiling_on_sc=False),
        out_type=jax.ShapeDtypeStruct((2, ACC_ROWS, fh), jnp.float32),
        scratch_types=[
            pltpu.VMEM((GC * CHUNK,), jnp.int32),
            pltpu.VMEM((GC, CHUNK), jnp.int32),
            pltpu.VMEM((GC, CHUNK, fh), jnp.float32),
            pltpu.VMEM_SHARED((ACC_ROWS, fh), jnp.float32),
            pltpu.SemaphoreType.DMA,
            pltpu.SemaphoreType.DMA,
            pltpu.SemaphoreType.DMA,
            pltpu.SemaphoreType.DMA,
            pltpu.SemaphoreType.DMA,
        ],
    )
    def _sc_segment_sum(table_hbm, src_hbm, didx3_hbm, didx_hbm, zeros_hbm,
                        out_hbm, si, di, rows, acc, sem_i, sg0, sg1, sg2, sg3):
        c = lax.axis_index("c")
        s = lax.axis_index("s")
        sg = [sg0, sg1, sg2, sg3]
        row0 = pl.multiple_of(s * ROWS_PER_TILE, 8)

        pltpu.sync_copy(zeros_hbm.at[pl.ds(row0, ROWS_PER_TILE)],
                        acc.at[pl.ds(row0, ROWS_PER_TILE)])
        plsc.subcore_barrier()

        def group(m, carry):
            ch0 = s * CPT + m * GC
            e0 = pl.multiple_of(ch0 * CHUNK, 8)
            h_si = pltpu.async_copy(src_hbm.at[pl.ds(e0, GC * CHUNK)],
                                    si, sem_i)
            h_di = pltpu.async_copy(didx3_hbm.at[s * NGRP + m], di, sem_i)
            h_si.wait()
            h_di.wait()
            hs = [pltpu.async_copy(
                      table_hbm.at[c].at[si.at[pl.ds(b * CHUNK, CHUNK)]],
                      rows.at[b], sg[b])
                  for b in range(GC)]
            for b in range(GC):
                hs[b].wait()
                pltpu.sync_copy(rows.at[b], acc.at[di.at[b]], add=True)
            return carry

        lax.fori_loop(0, NGRP, group, 0)

        @pl.when(s < NCHUNK - CPT * 16)
        def _():
            ch = 16 * CPT + s
            e0 = pl.multiple_of(ch * CHUNK, 8)
            pltpu.sync_copy(src_hbm.at[pl.ds(e0, CHUNK)],
                            si.at[pl.ds(0, CHUNK)])
            pltpu.sync_copy(didx_hbm.at[ch], di.at[0])
            pltpu.async_copy(table_hbm.at[c].at[si.at[pl.ds(0, CHUNK)]],
                             rows.at[0], sg0).wait()
            pltpu.sync_copy(rows.at[0], acc.at[di.at[0]], add=True)

        plsc.subcore_barrier()
        pltpu.sync_copy(acc.at[pl.ds(row0, ROWS_PER_TILE)],
                        out_hbm.at[c, pl.ds(row0, ROWS_PER_TILE)])

    return _sc_segment_sum


def _mlp1_body(x_ref, agg_ref, vec_ref, w1a_ref, w1b_ref, h_ref):
    v = vec_ref[...]
    a = agg_ref[...]
    agg = jnp.concatenate([a[0, 0] + a[0, 1], a[1, 0] + a[1, 1]], axis=1)
    t = x_ref[...] + agg
    t = jnp.dot(t, w1a_ref[...], preferred_element_type=jnp.float32) + v[0]
    t = t / jnp.sqrt(1.0 + 1e-5) * v[1] + v[2]
    t = jnp.maximum(t, 0.0)
    t = jnp.dot(t, w1b_ref[...], preferred_element_type=jnp.float32) + v[3]
    t = jnp.maximum(t, 0.0)
    h_ref[...] = t / jnp.sqrt(1.0 + 1e-5) * v[4] + v[5]


def _tc_mlp1(x, aggp, vecs, w1a, w1b):
    return pl.pallas_call(
        _mlp1_body,
        grid=(N // BLK,),
        in_specs=[
            pl.BlockSpec((BLK, F_IN), lambda i: (i, 0)),
            pl.BlockSpec((2, NCOPY, BLK, H), lambda i: (0, 0, i, 0)),
            pl.BlockSpec((8, H), lambda i: (0, 0)),
            pl.BlockSpec((F_IN, H), lambda i: (0, 0)),
            pl.BlockSpec((H, H), lambda i: (0, 0)),
        ],
        out_specs=pl.BlockSpec((BLK, H), lambda i: (i, 0)),
        out_shape=jax.ShapeDtypeStruct((N, H), jnp.float32),
    )(x, aggp, vecs, w1a, w1b)


def _final_body(h_ref, agg_ref, vec_ref, w2a_ref, w2b_ref, batch_ref,
                wl1_ref, wl2p_ref, out_ref, pool_acc):
    i = pl.program_id(0)
    v = vec_ref[...]
    a = agg_ref[...]
    agg = jnp.concatenate([a[0, 0] + a[0, 1], a[1, 0] + a[1, 1]], axis=1)
    t = h_ref[...] + agg
    t = jnp.dot(t, w2a_ref[...], preferred_element_type=jnp.float32) + v[0]
    t = t / jnp.sqrt(1.0 + 1e-5) * v[1] + v[2]
    t = jnp.maximum(t, 0.0)
    t = jnp.dot(t, w2b_ref[...], preferred_element_type=jnp.float32) + v[3]
    t = jnp.maximum(t, 0.0)
    h2 = t / jnp.sqrt(1.0 + 1e-5) * v[4] + v[5]

    gid = lax.broadcasted_iota(jnp.int32, (BLK, G), 1)
    onehot = jnp.where(batch_ref[...] == gid, 1.0, 0.0).astype(jnp.float32)
    part = lax.dot_general(onehot, h2, (((0,), (0,)), ((), ())),
                           preferred_element_type=jnp.float32,
                           precision=lax.Precision.HIGHEST)

    @pl.when(i == 0)
    def _():
        pool_acc[...] = jnp.zeros_like(pool_acc)

    pool_acc[...] += part

    @pl.when(i == pl.num_programs(0) - 1)
    def _():
        pooled = pool_acc[...]
        p1 = jnp.dot(pooled, wl1_ref[...],
                     preferred_element_type=jnp.float32) + v[6]
        p1 = jnp.maximum(p1, 0.0)
        out = jnp.dot(p1, wl2p_ref[...], preferred_element_type=jnp.float32)
        out_ref[...] = out[:, 0:1] + v[8][0]


def _tc_final(h, aggp, vecs, w2a, w2b, batch2d, wl1, wl2p):
    return pl.pallas_call(
        _final_body,
        grid=(N // BLK,),
        in_specs=[
            pl.BlockSpec((BLK, H), lambda i: (i, 0)),
            pl.BlockSpec((2, NCOPY, BLK, H // 2), lambda i: (0, 0, i, 0)),
            pl.BlockSpec((16, H), lambda i: (0, 0)),
            pl.BlockSpec((H, H), lambda i: (0, 0)),
            pl.BlockSpec((H, H), lambda i: (0, 0)),
            pl.BlockSpec((BLK, 1), lambda i: (i, 0)),
            pl.BlockSpec((H, G), lambda i: (0, 0)),
            pl.BlockSpec((H, 128), lambda i: (0, 0)),
        ],
        out_specs=pl.BlockSpec((G, 1), lambda i: (0, 0)),
        out_shape=jax.ShapeDtypeStruct((G, 1), jnp.float32),
        scratch_shapes=[pltpu.VMEM((G, G), jnp.float32)],
    )(h, aggp, vecs, w2a, w2b, batch2d, wl1, wl2p)


def kernel(x, edge_index, batch,
           w1a, b1a, g1a, be1a, w1b, b1b, g_bn1, be_bn1,
           w2a, b2a, g2a, be2a, w2b, b2b, g_bn2, be_bn2,
           wl1, bl1, wl2, bl2):
    src = edge_index[0]
    dst = edge_index[1]

    dst2 = jnp.concatenate(
        [dst, jnp.zeros((NCHUNK_PAD * CHUNK - E,), jnp.int32)]
    ).reshape(NCHUNK_PAD, CHUNK)
    didx = _tc_didx(dst2)
    didx3 = didx[:16 * CPT].reshape(16 * NGRP, GC, CHUNK)

    vecs1 = jnp.stack([
        b1a, g1a, be1a, b1b, g_bn1, be_bn1,
        jnp.zeros((H,), jnp.float32), jnp.zeros((H,), jnp.float32)])
    vecs2 = jnp.stack([
        b2a, g2a, be2a, b2b, g_bn2, be_bn2,
        bl1, wl2[:, 0], jnp.full((H,), bl2[0], jnp.float32)]
        + [jnp.zeros((H,), jnp.float32)] * 7)

    seg64 = _make_sc_segment_sum(H)
    seg32 = _make_sc_segment_sum(H // 2)

    x_halves = jnp.stack([x[:, :H], x[:, H:]])
    zeros64 = jnp.zeros((ACC_ROWS, H), jnp.float32)
    agg1 = seg64(x_halves, src, didx3, didx, zeros64)
    h = _tc_mlp1(x, agg1.reshape(2, NCOPY, N_PAD, H), vecs1, w1a, w1b)

    h_halves = jnp.stack([h[:, :H // 2], h[:, H // 2:]])
    zeros32 = jnp.zeros((ACC_ROWS, H // 2), jnp.float32)
    agg2 = seg32(h_halves, src, didx3, didx, zeros32)
    wl2p = jnp.concatenate(
        [wl2, jnp.zeros((H, 127), jnp.float32)], axis=1)
    out = _tc_final(h, agg2.reshape(2, NCOPY, N_PAD, H // 2), vecs2,
                    w2a, w2b, batch.reshape(N, 1), wl1, wl2p)
    return out


# speedup vs baseline: 1.7354x; 1.0234x over previous
import functools

import jax
import jax.numpy as jnp
from jax import lax
from jax.experimental import pallas as pl
from jax.experimental.pallas import tpu as pltpu
from jax.experimental.pallas import tpu_sc as plsc

N = 10000
E = 320000
F_IN = 128
H = 64
G = 64

CHUNK = 128
NCHUNK = E // CHUNK
N_PAD = 10112
NCOPY = 2
ACC_ROWS = NCOPY * N_PAD
ROWS_PER_TILE = ACC_ROWS // 16

CB = 64
NCHUNK_PAD = 2560
IDX_GRID = NCHUNK_PAD // CB

BLK = 2000


def _didx_body(dst_ref, out_ref):
    d = dst_ref[...]
    a = d[:, :, None]
    b = d[:, None, :]
    ii = lax.broadcasted_iota(jnp.int32, (CB, CHUNK, CHUNK), 1)
    jj = lax.broadcasted_iota(jnp.int32, (CB, CHUNK, CHUNK), 2)
    dup = jnp.logical_and(a == b, jj < ii)
    has = jnp.max(dup.astype(jnp.int32), axis=2)
    out_ref[...] = d + has * N_PAD


def _tc_didx(dst2):
    return pl.pallas_call(
        _didx_body,
        grid=(IDX_GRID,),
        in_specs=[pl.BlockSpec((CB, CHUNK), lambda i: (i, 0))],
        out_specs=pl.BlockSpec((CB, CHUNK), lambda i: (i, 0)),
        out_shape=jax.ShapeDtypeStruct((NCHUNK_PAD, CHUNK), jnp.int32),
    )(dst2)


CPT = NCHUNK // 16


def _gc(fh):
    return 4 if fh == H else 6


@functools.cache
def _make_sc_segment_sum(fh):
    GC = _gc(fh)
    NGRP = CPT // GC
    mesh = plsc.VectorSubcoreMesh(core_axis_name="c", subcore_axis_name="s")

    @functools.partial(
        pl.kernel,
        mesh=mesh,
        compiler_params=pltpu.CompilerParams(use_tc_tiling_on_sc=False),
        out_type=jax.ShapeDtypeStruct((2, ACC_ROWS, fh), jnp.float32),
        scratch_types=[
            pltpu.VMEM((GC * CHUNK,), jnp.int32),
            pltpu.VMEM((GC, CHUNK), jnp.int32),
            pltpu.VMEM((GC, CHUNK, fh), jnp.float32),
            pltpu.VMEM_SHARED((ACC_ROWS, fh), jnp.float32),
            pltpu.SemaphoreType.DMA,
            pltpu.SemaphoreType.DMA((GC,)),
        ],
    )
    def _sc_segment_sum(table_hbm, src_hbm, didx3_hbm, didx_hbm, zeros_hbm,
                        out_hbm, si, di, rows, acc, sem_i, sgs):
        c = lax.axis_index("c")
        s = lax.axis_index("s")
        sg = [sgs.at[b] for b in range(GC)]
        row0 = pl.multiple_of(s * ROWS_PER_TILE, 8)

        pltpu.sync_copy(zeros_hbm.at[pl.ds(row0, ROWS_PER_TILE)],
                        acc.at[pl.ds(row0, ROWS_PER_TILE)])
        plsc.subcore_barrier()

        def group(m, carry):
            ch0 = s * CPT + m * GC
            e0 = pl.multiple_of(ch0 * CHUNK, 8)
            h_si = pltpu.async_copy(src_hbm.at[pl.ds(e0, GC * CHUNK)],
                                    si, sem_i)
            h_di = pltpu.async_copy(didx3_hbm.at[s * NGRP + m], di, sem_i)
            h_si.wait()
            h_di.wait()
            hs = [pltpu.async_copy(
                      table_hbm.at[c].at[si.at[pl.ds(b * CHUNK, CHUNK)]],
                      rows.at[b], sg[b])
                  for b in range(GC)]
            for b in range(GC):
                hs[b].wait()
                pltpu.sync_copy(rows.at[b], acc.at[di.at[b]], add=True)
            return carry

        lax.fori_loop(0, NGRP, group, 0)

        @pl.when(s < NCHUNK - CPT * 16)
        def _():
            ch = 16 * CPT + s
            e0 = pl.multiple_of(ch * CHUNK, 8)
            pltpu.sync_copy(src_hbm.at[pl.ds(e0, CHUNK)],
                            si.at[pl.ds(0, CHUNK)])
            pltpu.sync_copy(didx_hbm.at[ch], di.at[0])
            pltpu.async_copy(table_hbm.at[c].at[si.at[pl.ds(0, CHUNK)]],
                             rows.at[0], sg[0]).wait()
            pltpu.sync_copy(rows.at[0], acc.at[di.at[0]], add=True)

        plsc.subcore_barrier()
        pltpu.sync_copy(acc.at[pl.ds(row0, ROWS_PER_TILE)],
                        out_hbm.at[c, pl.ds(row0, ROWS_PER_TILE)])

    return _sc_segment_sum


def _mlp1_body(x_ref, agg_ref, vec_ref, w1a_ref, w1b_ref, h_ref):
    v = vec_ref[...]
    a = agg_ref[...]
    agg = jnp.concatenate([a[0, 0] + a[0, 1], a[1, 0] + a[1, 1]], axis=1)
    t = x_ref[...] + agg
    t = jnp.dot(t, w1a_ref[...], preferred_element_type=jnp.float32) + v[0]
    t = t / jnp.sqrt(1.0 + 1e-5) * v[1] + v[2]
    t = jnp.maximum(t, 0.0)
    t = jnp.dot(t, w1b_ref[...], preferred_element_type=jnp.float32) + v[3]
    t = jnp.maximum(t, 0.0)
    h_ref[...] = t / jnp.sqrt(1.0 + 1e-5) * v[4] + v[5]


def _tc_mlp1(x, aggp, vecs, w1a, w1b):
    return pl.pallas_call(
        _mlp1_body,
        grid=(N // BLK,),
        in_specs=[
            pl.BlockSpec((BLK, F_IN), lambda i: (i, 0)),
            pl.BlockSpec((2, NCOPY, BLK, H), lambda i: (0, 0, i, 0)),
            pl.BlockSpec((8, H), lambda i: (0, 0)),
            pl.BlockSpec((F_IN, H), lambda i: (0, 0)),
            pl.BlockSpec((H, H), lambda i: (0, 0)),
        ],
        out_specs=pl.BlockSpec((BLK, H), lambda i: (i, 0)),
        out_shape=jax.ShapeDtypeStruct((N, H), jnp.float32),
    )(x, aggp, vecs, w1a, w1b)


def _final_body(h_ref, agg_ref, vec_ref, w2a_ref, w2b_ref, batch_ref,
                wl1_ref, wl2p_ref, out_ref, pool_acc):
    i = pl.program_id(0)
    v = vec_ref[...]
    a = agg_ref[...]
    agg = jnp.concatenate([a[0, 0] + a[0, 1], a[1, 0] + a[1, 1]], axis=1)
    t = h_ref[...] + agg
    t = jnp.dot(t, w2a_ref[...], preferred_element_type=jnp.float32) + v[0]
    t = t / jnp.sqrt(1.0 + 1e-5) * v[1] + v[2]
    t = jnp.maximum(t, 0.0)
    t = jnp.dot(t, w2b_ref[...], preferred_element_type=jnp.float32) + v[3]
    t = jnp.maximum(t, 0.0)
    h2 = t / jnp.sqrt(1.0 + 1e-5) * v[4] + v[5]

    gid = lax.broadcasted_iota(jnp.int32, (BLK, G), 1)
    onehot = jnp.where(batch_ref[...] == gid, 1.0, 0.0).astype(jnp.float32)
    part = lax.dot_general(onehot, h2, (((0,), (0,)), ((), ())),
                           preferred_element_type=jnp.float32,
                           precision=lax.Precision.HIGHEST)

    @pl.when(i == 0)
    def _():
        pool_acc[...] = jnp.zeros_like(pool_acc)

    pool_acc[...] += part

    @pl.when(i == pl.num_programs(0) - 1)
    def _():
        pooled = pool_acc[...]
        p1 = jnp.dot(pooled, wl1_ref[...],
                     preferred_element_type=jnp.float32) + v[6]
        p1 = jnp.maximum(p1, 0.0)
        out = jnp.dot(p1, wl2p_ref[...], preferred_element_type=jnp.float32)
        out_ref[...] = out[:, 0:1] + v[8][0]


def _tc_final(h, aggp, vecs, w2a, w2b, batch2d, wl1, wl2p):
    return pl.pallas_call(
        _final_body,
        grid=(N // BLK,),
        in_specs=[
            pl.BlockSpec((BLK, H), lambda i: (i, 0)),
            pl.BlockSpec((2, NCOPY, BLK, H // 2), lambda i: (0, 0, i, 0)),
            pl.BlockSpec((16, H), lambda i: (0, 0)),
            pl.BlockSpec((H, H), lambda i: (0, 0)),
            pl.BlockSpec((H, H), lambda i: (0, 0)),
            pl.BlockSpec((BLK, 1), lambda i: (i, 0)),
            pl.BlockSpec((H, G), lambda i: (0, 0)),
            pl.BlockSpec((H, 128), lambda i: (0, 0)),
        ],
        out_specs=pl.BlockSpec((G, 1), lambda i: (0, 0)),
        out_shape=jax.ShapeDtypeStruct((G, 1), jnp.float32),
        scratch_shapes=[pltpu.VMEM((G, G), jnp.float32)],
    )(h, aggp, vecs, w2a, w2b, batch2d, wl1, wl2p)


def kernel(x, edge_index, batch,
           w1a, b1a, g1a, be1a, w1b, b1b, g_bn1, be_bn1,
           w2a, b2a, g2a, be2a, w2b, b2b, g_bn2, be_bn2,
           wl1, bl1, wl2, bl2):
    src = edge_index[0]
    dst = edge_index[1]

    dst2 = jnp.concatenate(
        [dst, jnp.zeros((NCHUNK_PAD * CHUNK - E,), jnp.int32)]
    ).reshape(NCHUNK_PAD, CHUNK)
    didx = _tc_didx(dst2)
    body = didx[:16 * CPT]
    didx3_64 = body.reshape(16 * (CPT // _gc(H)), _gc(H), CHUNK)
    didx3_32 = body.reshape(16 * (CPT // _gc(H // 2)), _gc(H // 2), CHUNK)

    vecs1 = jnp.stack([
        b1a, g1a, be1a, b1b, g_bn1, be_bn1,
        jnp.zeros((H,), jnp.float32), jnp.zeros((H,), jnp.float32)])
    vecs2 = jnp.stack([
        b2a, g2a, be2a, b2b, g_bn2, be_bn2,
        bl1, wl2[:, 0], jnp.full((H,), bl2[0], jnp.float32)]
        + [jnp.zeros((H,), jnp.float32)] * 7)

    seg64 = _make_sc_segment_sum(H)
    seg32 = _make_sc_segment_sum(H // 2)

    x_halves = jnp.stack([x[:, :H], x[:, H:]])
    zeros64 = jnp.zeros((ACC_ROWS, H), jnp.float32)
    agg1 = seg64(x_halves, src, didx3_64, didx, zeros64)
    h = _tc_mlp1(x, agg1.reshape(2, NCOPY, N_PAD, H), vecs1, w1a, w1b)

    h_halves = jnp.stack([h[:, :H // 2], h[:, H // 2:]])
    zeros32 = jnp.zeros((ACC_ROWS, H // 2), jnp.float32)
    agg2 = seg32(h_halves, src, didx3_32, didx, zeros32)
    wl2p = jnp.concatenate(
        [wl2, jnp.zeros((H, 127), jnp.float32)], axis=1)
    out = _tc_final(h, agg2.reshape(2, NCOPY, N_PAD, H // 2), vecs2,
                    w2a, w2b, batch.reshape(N, 1), wl1, wl2p)
    return out
